# baseline (speedup 1.0000x reference)
import functools

import jax
import jax.numpy as jnp
from jax import lax
from jax.experimental import pallas as pl
from jax.experimental.pallas import tpu as pltpu
from jax.experimental.pallas import tpu_sc as plsc

B, V, C, H, W = 2, 8, 64, 128, 128
HW = H * W
S = V * HW
A = 512
P = A * A
NG = 8
GC = C // NG
NC, NS, L = 2, 16, 16
HALF = P // NC
NQ = 4
HQ = HALF // NQ
QPT = HQ // NS
SPT = S // NS
CH = 2048
NCHUNK = SPT // CH
SLAB = 128
NSLAB = CH // SLAB
BLK = 8192
HB = 2048


def _round_half_even(t):
    i = t.astype(jnp.int32)
    f = t - i.astype(jnp.float32)
    up = (f > 0.5) | ((f == 0.5) & ((i & 1) == 1))
    return i + jnp.where(up, 1, 0)


def _pre_body(a_ref, m_ref, i_ref, o_ref):
    a = a_ref[0, 0]
    m = m_ref[0, 0, 0]
    w = jnp.maximum(i_ref[0, 0, 0, 0], 0.0)
    wv = jnp.where(m > 0.5, w, 0.0)
    o_ref[0, 0] = (a * wv[None, :]).T


def _pre_scale(encoded, msk3, imp4):
    return pl.pallas_call(
        _pre_body,
        grid=(B, V, NG, HW // HB),
        in_specs=[
            pl.BlockSpec((1, 1, GC, HB), lambda b, v, g, h: (b, v, g, h)),
            pl.BlockSpec((1, 1, 1, HB), lambda b, v, g, h: (b, v, 0, h)),
            pl.BlockSpec((1, 1, 1, 1), lambda b, v, g, h: (b, v, 0, 0)),
        ],
        out_specs=pl.BlockSpec(
            (1, 1, HB, GC), lambda b, v, g, h: (b, g, v * (HW // HB) + h, 0)),
        out_shape=jax.ShapeDtypeStruct((B, NG, S, GC), jnp.float32),
    )(encoded, msk3, imp4)


def _sc_body(ev, uv, msk, imp, zrs, atlas_out, counts_out,
             uvbuf, maskbuf, impbuf, idxbuf, wbuf, featbuf, zbuf, zflat,
             atlas_sh, counts_sh, ldsem0, ldsem1, scsem0, scsem1, csem):
    c = lax.axis_index("c")
    s = lax.axis_index("s")
    base = s * SPT
    vid = base // HW
    iota = lax.iota(jnp.int32, L)
    iota2 = iota * 2
    zero16 = jnp.zeros((L,), jnp.float32)
    ldsems = [ldsem0, ldsem1]
    scsems = [scsem0, scsem1]

    @pl.loop(0, SPT // L)
    def _(i):
        zflat[pl.ds(i * L, L)] = zero16

    pltpu.sync_copy(zrs, zbuf)

    for q in range(QPT // CH):
        pltpu.sync_copy(zbuf, atlas_sh.at[pl.ds(s * QPT + q * CH, CH), :])
    pltpu.sync_copy(zflat.at[pl.ds(0, QPT)], counts_sh.at[pl.ds(s * QPT, QPT)])

    @pl.when(s == 0)
    def _():
        pltpu.sync_copy(zbuf.at[pl.ds(0, 8), :], atlas_sh.at[pl.ds(HQ, 8), :])
        pltpu.sync_copy(zflat.at[pl.ds(0, 8)], counts_sh.at[pl.ds(HQ, 8)])

    plsc.subcore_barrier()

    for b in range(B):
        pltpu.sync_copy(uv.at[b, pl.ds(base * 2, 2 * SPT)], uvbuf)
        pltpu.sync_copy(msk.at[b, pl.ds(base, SPT)], maskbuf)
        pltpu.sync_copy(imp.at[b], impbuf)
        wimp = plsc.load_gather(impbuf, [jnp.full((L,), vid, jnp.int32)])
        wimp = jnp.maximum(wimp, 0.0)

        for qq in range(NQ):
            lo = c * HALF + qq * HQ

            @pl.loop(0, SPT // L)
            def _(i):
                u = plsc.load_gather(uvbuf, [iota2 + i * (2 * L)])
                v = plsc.load_gather(uvbuf, [iota2 + (i * (2 * L) + 1)])
                m = maskbuf[pl.ds(i * L, L)]
                x = _round_half_even(jnp.clip(u, 0.0, 1.0) * float(A - 1))
                y = _round_half_even(
                    (1.0 - jnp.clip(v, 0.0, 1.0)) * float(A - 1))
                lidx = y * A + x - lo
                in_rng = (lidx >= 0) & (lidx < HQ)
                wbuf[pl.ds(i * L, L)] = jnp.where(
                    (m > 0.5) & (wimp > 0.0), wimp, 0.0)
                idxbuf[i // (SLAB // L), pl.ds((i % (SLAB // L)) * L, L)] = (
                    jnp.where(in_rng, lidx, HQ))

            @pl.loop(0, NG)
            def _(g):
                scat = [[], []]
                loads = [None, None]
                loads[0] = pltpu.async_copy(
                    ev.at[b, g, pl.ds(base, CH), :], featbuf.at[0], ldsems[0])
                for k in range(NCHUNK):
                    cur = k % 2
                    nxt = (k + 1) % 2
                    if k + 1 < NCHUNK:
                        for d in scat[nxt]:
                            d.wait()
                        scat[nxt] = []
                        loads[nxt] = pltpu.async_copy(
                            ev.at[b, g, pl.ds(base + (k + 1) * CH, CH), :],
                            featbuf.at[nxt], ldsems[nxt])
                    loads[cur].wait()
                    for j in range(NSLAB):
                        scat[cur].append(pltpu.async_copy(
                            featbuf.at[cur, pl.ds(j * SLAB, SLAB), :],
                            atlas_sh.at[idxbuf.at[k * NSLAB + j]],
                            scsems[cur], add=True))

                @pl.when(g == 0)
                def _():
                    cds = [pltpu.async_copy(
                        wbuf.at[pl.ds(j * SLAB, SLAB)],
                        counts_sh.at[idxbuf.at[j]], csem, add=True)
                        for j in range(SPT // SLAB)]
                    for d in cds:
                        d.wait()

                for q in range(2):
                    for d in scat[q]:
                        d.wait()
                pltpu.async_copy(
                    atlas_sh.at[idxbuf.at[NCHUNK * NSLAB - 1]],
                    featbuf.at[0, pl.ds(0, SLAB), :], ldsems[0]).wait()
                plsc.subcore_barrier()

                pltpu.sync_copy(
                    atlas_sh.at[pl.ds(s * QPT, QPT), :],
                    atlas_out.at[b, g, pl.ds(lo + s * QPT, QPT), :])

                @pl.when(g == 0)
                def _():
                    pltpu.sync_copy(
                        counts_sh.at[pl.ds(s * QPT, QPT)],
                        counts_out.at[b, pl.ds(lo + s * QPT, QPT)])
                    pltpu.sync_copy(zflat.at[pl.ds(0, QPT)],
                                    counts_sh.at[pl.ds(s * QPT, QPT)])

                for q in range(QPT // CH):
                    pltpu.sync_copy(
                        zbuf, atlas_sh.at[pl.ds(s * QPT + q * CH, CH), :])
                plsc.subcore_barrier()


_sc_fuse = functools.partial(
    pl.kernel,
    out_type=(jax.ShapeDtypeStruct((B, NG, P, GC), jnp.float32),
              jax.ShapeDtypeStruct((B, P), jnp.float32)),
    mesh=plsc.VectorSubcoreMesh(core_axis_name="c", subcore_axis_name="s"),
    compiler_params=pltpu.CompilerParams(
        needs_layout_passes=False, use_tc_tiling_on_sc=False),
    scratch_types=[
        pltpu.VMEM((2 * SPT,), jnp.float32),
        pltpu.VMEM((SPT,), jnp.float32),
        pltpu.VMEM((L,), jnp.float32),
        pltpu.VMEM((SPT // SLAB, SLAB), jnp.int32),
        pltpu.VMEM((SPT,), jnp.float32),
        pltpu.VMEM((2, CH, GC), jnp.float32),
        pltpu.VMEM((CH, GC), jnp.float32),
        pltpu.VMEM((SPT,), jnp.float32),
        pltpu.VMEM_SHARED((HQ + 8, GC), jnp.float32),
        pltpu.VMEM_SHARED((HQ + 8,), jnp.float32),
        pltpu.SemaphoreType.DMA,
        pltpu.SemaphoreType.DMA,
        pltpu.SemaphoreType.DMA,
        pltpu.SemaphoreType.DMA,
        pltpu.SemaphoreType.DMA,
    ],
)(_sc_body)


def _tc_body(a_ref, c_ref, o_ref, v_ref):
    a = a_ref[0, 0]
    cnt = c_ref[0, 0]
    recip = 1.0 / jnp.maximum(cnt, 1.0)
    o_ref[0] = a.T * recip[None, :]
    v_ref[0] = (cnt > 0.0).astype(jnp.float32)[None]


def _tc_finish(atlas_s, counts):
    fused_flat, val_flat = pl.pallas_call(
        _tc_body,
        grid=(B, NG, P // BLK),
        in_specs=[
            pl.BlockSpec((1, 1, BLK, GC), lambda b, g, k: (b, g, k, 0)),
            pl.BlockSpec((1, 1, BLK), lambda b, g, k: (b * (P // BLK) + k, 0, 0)),
        ],
        out_specs=[
            pl.BlockSpec((1, NG, BLK), lambda b, g, k: (b, g, k)),
            pl.BlockSpec((1, 1, BLK), lambda b, g, k: (b, 0, k)),
        ],
        out_shape=[
            jax.ShapeDtypeStruct((B, C, P), jnp.float32),
            jax.ShapeDtypeStruct((B, 1, P), jnp.float32),
        ],
    )(atlas_s, counts.reshape(B * (P // BLK), 1, BLK))
    return (fused_flat.reshape(B, C, A, A),
            val_flat.reshape(B, 1, A, A))


def kernel(encoded_views, atlas_size, view_uvs, view_masks, view_importance):
    ev_t = _pre_scale(encoded_views.reshape(B, V, C, HW),
                      view_masks.reshape(B, V, 1, HW),
                      view_importance.reshape(B, V, 1, 1).astype(jnp.float32))
    uv = view_uvs.reshape(B, 2 * S)
    msk = view_masks.reshape(B, S)
    imp = jnp.pad(view_importance.astype(jnp.float32), ((0, 0), (0, L - V)))
    zrs = jnp.zeros((CH, GC), jnp.float32)
    atlas_s, counts = _sc_fuse(ev_t, uv, msk, imp, zrs)
    return _tc_finish(atlas_s, counts)


# speedup vs baseline: 2.4495x; 2.4495x over previous
import functools

import jax
import jax.numpy as jnp
from jax import lax
from jax.experimental import pallas as pl
from jax.experimental.pallas import tpu as pltpu
from jax.experimental.pallas import tpu_sc as plsc

B, V, C, H, W = 2, 8, 64, 128, 128
HW = H * W
S = V * HW
A = 512
P = A * A
NG = 8
GC = C // NG
NC, NS, L = 2, 16, 16
HALF = P // NC
NQ = 1
HQ = HALF // NQ
QPT = HQ // NS
SPT = S // NS
CH = 512
NCHUNK = SPT // CH
SLAB = 128
NSLAB = CH // SLAB
ZF = 2048
BLK = 8192
HB = 2048


def _round_half_even(t):
    i = t.astype(jnp.int32)
    f = t - i.astype(jnp.float32)
    up = (f > 0.5) | ((f == 0.5) & ((i & 1) == 1))
    return i + jnp.where(up, 1, 0)


def _pre_body(a_ref, m_ref, i_ref, o_ref):
    a = a_ref[0, 0]
    m = m_ref[0, 0, 0]
    w = jnp.maximum(i_ref[0, 0, 0, 0], 0.0)
    wv = jnp.where(m > 0.5, w, 0.0)
    o_ref[0, 0] = (a * wv[None, :]).T


def _pre_scale(encoded, msk3, imp4):
    return pl.pallas_call(
        _pre_body,
        grid=(B, V, NG, HW // HB),
        in_specs=[
            pl.BlockSpec((1, 1, GC, HB), lambda b, v, g, h: (b, v, g, h)),
            pl.BlockSpec((1, 1, 1, HB), lambda b, v, g, h: (b, v, 0, h)),
            pl.BlockSpec((1, 1, 1, 1), lambda b, v, g, h: (b, v, 0, 0)),
        ],
        out_specs=pl.BlockSpec(
            (1, 1, HB, GC), lambda b, v, g, h: (b, g, v * (HW // HB) + h, 0)),
        out_shape=jax.ShapeDtypeStruct((B, NG, S, GC), jnp.float32),
    )(encoded, msk3, imp4)


def _sc_body(ev, uv, msk, imp, zrs, atlas_out, counts_out,
             uvbuf, maskbuf, impbuf, idxbuf, wbuf, featbuf, zbuf, zflat,
             atlas_sh, counts_sh, ldsem0, ldsem1, scsem0, scsem1, csem):
    c = lax.axis_index("c")
    s = lax.axis_index("s")
    base = s * SPT
    vid = base // HW
    iota = lax.iota(jnp.int32, L)
    iota2 = iota * 2
    zero16 = jnp.zeros((L,), jnp.float32)
    ldsems = [ldsem0, ldsem1]
    scsems = [scsem0, scsem1]

    @pl.loop(0, ZF // L)
    def _(i):
        zflat[pl.ds(i * L, L)] = zero16

    pltpu.sync_copy(zrs, zbuf)

    for q in range(QPT // CH):
        pltpu.sync_copy(zbuf, atlas_sh.at[pl.ds(s * QPT + q * CH, CH), :])
    for z in range(QPT // ZF):
        pltpu.sync_copy(zflat, counts_sh.at[pl.ds(s * QPT + z * ZF, ZF)])

    @pl.when(s == 0)
    def _():
        pltpu.sync_copy(zbuf.at[pl.ds(0, 8), :], atlas_sh.at[pl.ds(HQ, 8), :])
        pltpu.sync_copy(zflat.at[pl.ds(0, 8)], counts_sh.at[pl.ds(HQ, 8)])

    plsc.subcore_barrier()

    for b in range(B):
        pltpu.sync_copy(uv.at[b, pl.ds(base * 2, 2 * SPT)], uvbuf)
        pltpu.sync_copy(msk.at[b, pl.ds(base, SPT)], maskbuf)
        pltpu.sync_copy(imp.at[b], impbuf)
        wimp = plsc.load_gather(impbuf, [jnp.full((L,), vid, jnp.int32)])
        wimp = jnp.maximum(wimp, 0.0)

        for qq in range(NQ):
            lo = c * HALF + qq * HQ

            @pl.loop(0, SPT // L)
            def _(i):
                u = plsc.load_gather(uvbuf, [iota2 + i * (2 * L)])
                v = plsc.load_gather(uvbuf, [iota2 + (i * (2 * L) + 1)])
                m = maskbuf[pl.ds(i * L, L)]
                x = _round_half_even(jnp.clip(u, 0.0, 1.0) * float(A - 1))
                y = _round_half_even(
                    (1.0 - jnp.clip(v, 0.0, 1.0)) * float(A - 1))
                lidx = y * A + x - lo
                in_rng = (lidx >= 0) & (lidx < HQ)
                wbuf[pl.ds(i * L, L)] = jnp.where(
                    (m > 0.5) & (wimp > 0.0), wimp, 0.0)
                idxbuf[i // (SLAB // L), pl.ds((i % (SLAB // L)) * L, L)] = (
                    jnp.where(in_rng, lidx, HQ))

            @pl.loop(0, NG)
            def _(g):
                scat = [[], []]
                loads = [None, None]
                loads[0] = pltpu.async_copy(
                    ev.at[b, g, pl.ds(base, CH), :], featbuf.at[0], ldsems[0])
                for k in range(NCHUNK):
                    cur = k % 2
                    nxt = (k + 1) % 2
                    if k + 1 < NCHUNK:
                        for d in scat[nxt]:
                            d.wait()
                        scat[nxt] = []
                        loads[nxt] = pltpu.async_copy(
                            ev.at[b, g, pl.ds(base + (k + 1) * CH, CH), :],
                            featbuf.at[nxt], ldsems[nxt])
                    loads[cur].wait()
                    for j in range(NSLAB):
                        scat[cur].append(pltpu.async_copy(
                            featbuf.at[cur, pl.ds(j * SLAB, SLAB), :],
                            atlas_sh.at[idxbuf.at[k * NSLAB + j]],
                            scsems[cur], add=True))

                @pl.when(g == 0)
                def _():
                    cds = [pltpu.async_copy(
                        wbuf.at[pl.ds(j * SLAB, SLAB)],
                        counts_sh.at[idxbuf.at[j]], csem, add=True)
                        for j in range(SPT // SLAB)]
                    for d in cds:
                        d.wait()

                for q in range(2):
                    for d in scat[q]:
                        d.wait()
                pltpu.async_copy(
                    atlas_sh.at[idxbuf.at[NCHUNK * NSLAB - 1]],
                    featbuf.at[0, pl.ds(0, SLAB), :], ldsems[0]).wait()
                plsc.subcore_barrier()

                pltpu.sync_copy(
                    atlas_sh.at[pl.ds(s * QPT, QPT), :],
                    atlas_out.at[b, g, pl.ds(lo + s * QPT, QPT), :])

                @pl.when(g == 0)
                def _():
                    pltpu.sync_copy(
                        counts_sh.at[pl.ds(s * QPT, QPT)],
                        counts_out.at[b, pl.ds(lo + s * QPT, QPT)])
                    for z in range(QPT // ZF):
                        pltpu.sync_copy(
                            zflat, counts_sh.at[pl.ds(s * QPT + z * ZF, ZF)])

                for q in range(QPT // CH):
                    pltpu.sync_copy(
                        zbuf, atlas_sh.at[pl.ds(s * QPT + q * CH, CH), :])
                plsc.subcore_barrier()


_sc_fuse = functools.partial(
    pl.kernel,
    out_type=(jax.ShapeDtypeStruct((B, NG, P, GC), jnp.float32),
              jax.ShapeDtypeStruct((B, P), jnp.float32)),
    mesh=plsc.VectorSubcoreMesh(core_axis_name="c", subcore_axis_name="s"),
    compiler_params=pltpu.CompilerParams(
        needs_layout_passes=False, use_tc_tiling_on_sc=False),
    scratch_types=[
        pltpu.VMEM((2 * SPT,), jnp.float32),
        pltpu.VMEM((SPT,), jnp.float32),
        pltpu.VMEM((L,), jnp.float32),
        pltpu.VMEM((SPT // SLAB, SLAB), jnp.int32),
        pltpu.VMEM((SPT,), jnp.float32),
        pltpu.VMEM((2, CH, GC), jnp.float32),
        pltpu.VMEM((CH, GC), jnp.float32),
        pltpu.VMEM((ZF,), jnp.float32),
        pltpu.VMEM_SHARED((HQ + 8, GC), jnp.float32),
        pltpu.VMEM_SHARED((HQ + 8,), jnp.float32),
        pltpu.SemaphoreType.DMA,
        pltpu.SemaphoreType.DMA,
        pltpu.SemaphoreType.DMA,
        pltpu.SemaphoreType.DMA,
        pltpu.SemaphoreType.DMA,
    ],
)(_sc_body)


def _tc_body(a_ref, c_ref, o_ref, v_ref):
    a = a_ref[0, 0]
    cnt = c_ref[0, 0]
    recip = 1.0 / jnp.maximum(cnt, 1.0)
    o_ref[0] = a.T * recip[None, :]
    v_ref[0] = (cnt > 0.0).astype(jnp.float32)[None]


def _tc_finish(atlas_s, counts):
    fused_flat, val_flat = pl.pallas_call(
        _tc_body,
        grid=(B, NG, P // BLK),
        in_specs=[
            pl.BlockSpec((1, 1, BLK, GC), lambda b, g, k: (b, g, k, 0)),
            pl.BlockSpec((1, 1, BLK), lambda b, g, k: (b * (P // BLK) + k, 0, 0)),
        ],
        out_specs=[
            pl.BlockSpec((1, NG, BLK), lambda b, g, k: (b, g, k)),
            pl.BlockSpec((1, 1, BLK), lambda b, g, k: (b, 0, k)),
        ],
        out_shape=[
            jax.ShapeDtypeStruct((B, C, P), jnp.float32),
            jax.ShapeDtypeStruct((B, 1, P), jnp.float32),
        ],
    )(atlas_s, counts.reshape(B * (P // BLK), 1, BLK))
    return (fused_flat.reshape(B, C, A, A),
            val_flat.reshape(B, 1, A, A))


def kernel(encoded_views, atlas_size, view_uvs, view_masks, view_importance):
    ev_t = _pre_scale(encoded_views.reshape(B, V, C, HW),
                      view_masks.reshape(B, V, 1, HW),
                      view_importance.reshape(B, V, 1, 1).astype(jnp.float32))
    uv = view_uvs.reshape(B, 2 * S)
    msk = view_masks.reshape(B, S)
    imp = jnp.pad(view_importance.astype(jnp.float32), ((0, 0), (0, L - V)))
    zrs = jnp.zeros((CH, GC), jnp.float32)
    atlas_s, counts = _sc_fuse(ev_t, uv, msk, imp, zrs)
    return _tc_finish(atlas_s, counts)
